# baseline (speedup 1.0000x reference)
import jax
import jax.numpy as jnp
from jax.experimental import pallas as pl
from jax.experimental.pallas import tpu as pltpu

_B = 16
_NS = 4


def _eye(n, dtype):
    return (jax.lax.broadcasted_iota(jnp.int32, (n, n), 0)
            == jax.lax.broadcasted_iota(jnp.int32, (n, n), 1)).astype(dtype)


def _body(*refs):
    ids_ref = refs[0]
    x_refs = refs[1:1 + _NS]
    att_ref = refs[1 + _NS]
    bias_ref = refs[2 + _NS]
    out_refs = refs[3 + _NS:3 + 2 * _NS]
    m_ref, s_ref = refs[3 + 2 * _NS:]

    i = pl.program_id(0)
    nb = pl.num_programs(0)

    @pl.when(i == 0)
    def _init():
        m_ref[...] = jnp.full_like(m_ref, -jnp.inf)
        s_ref[...] = jnp.zeros_like(s_ref)
        for o in out_refs:
            o[...] = jnp.zeros_like(o)

    xs = [r[...].astype(jnp.bfloat16) for r in x_refs]
    att = att_ref[...].astype(jnp.bfloat16)
    hd = xs[0].shape[1]
    dn = (((1,), (0,)), ((), ()))
    l = sum(jax.lax.dot_general(x, att[j * hd:(j + 1) * hd], dn,
                                preferred_element_type=jnp.float32)
            for j, x in enumerate(xs))
    l = l + bias_ref[0, 0]
    ids = ids_ref[...]
    oh = ids == jax.lax.broadcasted_iota(jnp.int32, (1, _B), 1)

    m_old = m_ref[...]
    bm = jnp.max(jnp.where(oh, l, -jnp.inf), axis=0, keepdims=True)
    m_new = jnp.maximum(m_old, bm)
    scale = jnp.where(m_old == -jnp.inf, 0.0, jnp.exp(m_old - m_new))
    p = jnp.exp(jnp.where(oh, l - m_new, -jnp.inf))

    s_ref[...] = s_ref[...] * scale + jnp.sum(p, axis=0, keepdims=True)
    m_ref[...] = m_new

    eye = _eye(_B, jnp.float32)
    tdn = (((1,), (1,)), ((), ()))
    scale_col = jax.lax.dot_general(eye, scale, tdn,
                                    preferred_element_type=jnp.float32)
    ph = p.astype(jnp.bfloat16)
    cdn = (((0,), (0,)), ((), ()))
    for o, x in zip(out_refs, xs):
        o[...] = o[...] * scale_col + jax.lax.dot_general(
            ph, x, cdn, preferred_element_type=jnp.float32)

    @pl.when(i == nb - 1)
    def _fin():
        s_col = jax.lax.dot_general(eye, s_ref[...], tdn,
                                    preferred_element_type=jnp.float32)
        inv = jnp.where(s_col > 0.0, 1.0 / s_col, 0.0)
        for o in out_refs:
            o[...] = o[...] * inv


def _run(ids, flat, att2, bias2, blk):
    n, d = flat.shape
    hd = d // _NS

    def xspec(j):
        return pl.BlockSpec((blk, hd), lambda i, j=j: (i, j))

    outs = pl.pallas_call(
        _body,
        grid=(n // blk,),
        in_specs=(
            [pl.BlockSpec((blk, 1), lambda i: (i, 0))]
            + [xspec(j) for j in range(_NS)]
            + [pl.BlockSpec((d, 1), lambda i: (0, 0)),
               pl.BlockSpec((1, 1), lambda i: (0, 0))]
        ),
        out_specs=[pl.BlockSpec((_B, hd), lambda i: (0, 0))] * _NS,
        out_shape=[jax.ShapeDtypeStruct((_B, hd), jnp.float32)] * _NS,
        scratch_shapes=[
            pltpu.VMEM((1, _B), jnp.float32),
            pltpu.VMEM((1, _B), jnp.float32),
        ],
    )(ids, *([flat] * _NS), att2, bias2)
    return jnp.concatenate(outs, axis=1)


@jax.jit
def kernel(flat, segment_ids, att, bias, temperature):
    n, _ = flat.shape
    att2 = att * temperature[0]
    bias2 = (bias[0] * temperature[0]).reshape(1, 1)
    ids = segment_ids.astype(jnp.int32).reshape(n, 1)
    return _run(ids, flat, att2, bias2, 2048)
